# baseline (speedup 1.0000x reference)
import functools

import jax
import jax.numpy as jnp
from jax import lax
from jax.experimental import pallas as pl
from jax.experimental.pallas import tpu as pltpu
from jax.experimental.pallas import tpu_sc as plsc

N_NODES = 10000
NPAD = 10240
IN_CH = 256
HID = 512
NC = 2
NS = 16
NW = NC * NS
CHUNK = 128
EPAD = 163840
NQ = 4
QW = IN_CH // NQ
NSEG = NPAD // NS


def _sc_mesh():
    return plsc.VectorSubcoreMesh(
        core_axis_name="c", subcore_axis_name="s", num_cores=NC, num_subcores=NS
    )


def _sc_degree(dst2d):
    rows_w = EPAD // NW // CHUNK

    @functools.partial(
        pl.kernel,
        out_type=jax.ShapeDtypeStruct((NC * NPAD,), jnp.float32),
        mesh=_sc_mesh(),
        scratch_types=[
            pltpu.VMEM((rows_w, CHUNK), jnp.int32),
            pltpu.VMEM((CHUNK,), jnp.float32),
            pltpu.VMEM((NSEG,), jnp.float32),
            pltpu.VMEM_SHARED((NPAD,), jnp.float32),
            pltpu.SemaphoreType.DMA,
        ],
    )
    def k(dst_hbm, out_hbm, idx_v, ones_v, stage_v, acc, sem):
        c = lax.axis_index("c")
        s = lax.axis_index("s")
        w = c * NS + s
        ones16 = jnp.ones((16,), jnp.float32)
        zero16 = jnp.zeros((16,), jnp.float32)
        for j in range(CHUNK // 16):
            ones_v[pl.ds(j * 16, 16)] = ones16

        def zfill(i, carry):
            stage_v[pl.ds(i * 16, 16)] = zero16
            return carry

        lax.fori_loop(0, NSEG // 16, zfill, 0)
        pltpu.sync_copy(stage_v, acc.at[pl.ds(s * NSEG, NSEG)])
        pltpu.sync_copy(dst_hbm.at[pl.ds(w * rows_w, rows_w)], idx_v)
        plsc.subcore_barrier()

        def fire(j, carry):
            pltpu.async_copy(ones_v, acc.at[idx_v.at[j]], sem, add=True)
            return carry

        lax.fori_loop(0, rows_w, fire, 0)

        def drain(j, carry):
            pltpu.make_async_copy(ones_v, acc.at[idx_v.at[0]], sem).wait()
            return carry

        lax.fori_loop(0, rows_w, drain, 0)
        plsc.subcore_barrier()
        pltpu.sync_copy(acc.at[pl.ds(s * NSEG, NSEG)], stage_v)
        pltpu.sync_copy(stage_v, out_hbm.at[pl.ds(c * NPAD + s * NSEG, NSEG)])

    return k(dst2d)


def _sc_rowscatter(src2d, dst2d, xs_st):
    rows_t = EPAD // NS // CHUNK

    @functools.partial(
        pl.kernel,
        out_type=jax.ShapeDtypeStruct((NQ * NPAD, QW), jnp.float32),
        mesh=_sc_mesh(),
        scratch_types=[
            pltpu.VMEM((rows_t * CHUNK,), jnp.int32),
            pltpu.VMEM((rows_t, CHUNK), jnp.int32),
            pltpu.VMEM((2 * CHUNK, QW), jnp.float32),
            pltpu.VMEM((2 * CHUNK, QW), jnp.float32),
            pltpu.VMEM((CHUNK, QW), jnp.float32),
            pltpu.VMEM_SHARED((NPAD, QW), jnp.float32),
            pltpu.SemaphoreType.DMA,
            pltpu.SemaphoreType.DMA,
        ],
        compiler_params=pltpu.CompilerParams(use_tc_tiling_on_sc=False),
    )
    def k(src_hbm, dst_hbm, xs_hbm, out_hbm, srcb, dstb, rowb, rowb1, rowb2,
          acc, gs0, gs1):
        c = lax.axis_index("c")
        s = lax.axis_index("s")
        zero16 = jnp.zeros((16,), jnp.float32)
        coff = jnp.broadcast_to(c * (2 * NPAD), (16,)).astype(jnp.int32)
        noff = jnp.broadcast_to(NPAD, (16,)).astype(jnp.int32)

        pltpu.sync_copy(src_hbm.at[pl.ds(s * rows_t * CHUNK, rows_t * CHUNK)], srcb)
        pltpu.sync_copy(dst_hbm.at[pl.ds(s * rows_t, rows_t)], dstb)

        def shift(i, carry):
            sl = pl.ds(i * 16, 16)
            srcb[sl] = srcb[sl] + coff
            return carry

        lax.fori_loop(0, rows_t * CHUNK // 16, shift, 0)

        def shiftn(i, carry):
            sl = pl.ds(i * 16, 16)
            srcb[sl] = srcb[sl] + noff
            return carry

        npair = rows_t // 2

        pltpu.async_copy(xs_hbm.at[srcb.at[pl.ds(0, 2 * CHUNK)]], rowb, gs0)

        for q in range(2):
            if q == 0:
                bufs = (rowb, rowb1)
                gsems = (gs0, gs1)
            else:
                bufs = (rowb1, rowb)
                gsems = (gs1, gs0)
            zbuf = rowb2

            def zfill(i, carry):
                for j in range(QW // 16):
                    zbuf[i, pl.ds(j * 16, 16)] = zero16
                return carry

            lax.fori_loop(0, CHUNK, zfill, 0)

            def zacc(i, carry):
                pltpu.sync_copy(zbuf, acc.at[pl.ds(s * NSEG + i * CHUNK, CHUNK)])
                return carry

            lax.fori_loop(0, NSEG // CHUNK, zacc, 0)
            plsc.subcore_barrier()

            def gwait(b):
                pltpu.make_async_copy(
                    xs_hbm.at[srcb.at[pl.ds(0, 2 * CHUNK)]], bufs[b],
                    gsems[b]).wait()

            def body(jj, carry):
                for b in range(2):
                    p = jj * 2 + b
                    gwait(b)
                    pn = lax.rem(p + 1, npair)
                    pltpu.async_copy(
                        xs_hbm.at[srcb.at[pl.ds(2 * CHUNK * pn, 2 * CHUNK)]],
                        bufs[1 - b], gsems[1 - b])
                    pltpu.sync_copy(bufs[b].at[pl.ds(0, CHUNK)],
                                    acc.at[dstb.at[2 * p]], add=True)
                    pltpu.sync_copy(bufs[b].at[pl.ds(CHUNK, CHUNK)],
                                    acc.at[dstb.at[2 * p + 1]], add=True)
                return carry

            lax.fori_loop(0, npair // 2, body, 0)
            gwait(0)
            plsc.subcore_barrier()

            if q == 0:
                lax.fori_loop(0, rows_t * CHUNK // 16, shiftn, 0)
                pltpu.async_copy(xs_hbm.at[srcb.at[pl.ds(0, 2 * CHUNK)]], rowb1, gs1)

            def drain(i, carry):
                sl = pl.ds(s * NSEG + i * CHUNK, CHUNK)
                pltpu.sync_copy(acc.at[sl], zbuf)
                pltpu.sync_copy(
                    zbuf,
                    out_hbm.at[pl.ds((2 * c + q) * NPAD + s * NSEG + i * CHUNK, CHUNK)],
                )
                return carry

            lax.fori_loop(0, NSEG // CHUNK, drain, 0)

    return k(src2d, dst2d, xs_st)


def _sc_scalar_scatter(src2d, dst2d, ys):
    rows_w = EPAD // NW // CHUNK

    @functools.partial(
        pl.kernel,
        out_type=jax.ShapeDtypeStruct((NC * NPAD,), jnp.float32),
        mesh=_sc_mesh(),
        scratch_types=[
            pltpu.VMEM((rows_w, CHUNK), jnp.int32),
            pltpu.VMEM((rows_w, CHUNK), jnp.int32),
            pltpu.VMEM((rows_w, CHUNK), jnp.float32),
            pltpu.VMEM((NSEG,), jnp.float32),
            pltpu.VMEM_SHARED((NPAD,), jnp.float32),
            pltpu.SemaphoreType.DMA,
        ],
    )
    def k(src_hbm, dst_hbm, ys_hbm, out_hbm, srcb, dstb, vals, stage_v, acc, sem):
        c = lax.axis_index("c")
        s = lax.axis_index("s")
        w = c * NS + s
        zero16 = jnp.zeros((16,), jnp.float32)

        def zfill(i, carry):
            stage_v[pl.ds(i * 16, 16)] = zero16
            return carry

        lax.fori_loop(0, NSEG // 16, zfill, 0)
        pltpu.sync_copy(stage_v, acc.at[pl.ds(s * NSEG, NSEG)])
        pltpu.sync_copy(src_hbm.at[pl.ds(w * rows_w, rows_w)], srcb)
        pltpu.sync_copy(dst_hbm.at[pl.ds(w * rows_w, rows_w)], dstb)

        def gfire(j, carry):
            pltpu.async_copy(ys_hbm.at[srcb.at[j]], vals.at[j], sem)
            return carry

        lax.fori_loop(0, rows_w, gfire, 0)

        def gdrain(j, carry):
            pltpu.make_async_copy(ys_hbm.at[srcb.at[0]], vals.at[0], sem).wait()
            return carry

        lax.fori_loop(0, rows_w, gdrain, 0)
        plsc.subcore_barrier()

        def sfire(j, carry):
            pltpu.async_copy(vals.at[j], acc.at[dstb.at[j]], sem, add=True)
            return carry

        lax.fori_loop(0, rows_w, sfire, 0)

        def sdrain(j, carry):
            pltpu.make_async_copy(vals.at[0], acc.at[dstb.at[0]], sem).wait()
            return carry

        lax.fori_loop(0, rows_w, sdrain, 0)
        plsc.subcore_barrier()
        pltpu.sync_copy(acc.at[pl.ds(s * NSEG, NSEG)], stage_v)
        pltpu.sync_copy(stage_v, out_hbm.at[pl.ds(c * NPAD + s * NSEG, NSEG)])

    return k(src2d, dst2d, ys)


def _tc_prep(degp, x_pad):
    BM = 1024
    nb = NPAD // BM

    def body(dp0_ref, dp1_ref, x_ref, dinv_ref, xs_ref):
        deg = 1.0 + dp0_ref[...] + dp1_ref[...]
        dinv = lax.rsqrt(deg)
        dinv_ref[...] = dinv
        for q in range(NQ):
            xs_ref[q] = x_ref[:, pl.ds(q * QW, QW)] * dinv

    dinv, xs4 = pl.pallas_call(
        body,
        grid=(nb,),
        in_specs=[
            pl.BlockSpec((BM, 1), lambda i: (i, 0)),
            pl.BlockSpec((BM, 1), lambda i: (nb + i, 0)),
            pl.BlockSpec((BM, IN_CH), lambda i: (i, 0)),
        ],
        out_specs=[
            pl.BlockSpec((BM, 1), lambda i: (i, 0)),
            pl.BlockSpec((NQ, BM, QW), lambda i: (0, i, 0)),
        ],
        out_shape=[
            jax.ShapeDtypeStruct((NPAD, 1), jnp.float32),
            jax.ShapeDtypeStruct((NQ, NPAD, QW), jnp.float32),
        ],
    )(degp, degp, x_pad)
    return dinv, xs4.reshape(NQ * NPAD, QW)


def _tc_mlp(p0_st, xs_st, dinv, W1, b1, W2):
    BM = 512
    nb = NPAD // BM

    def body(p0a, p0b, p0c, p0d, xsa, xsb, xsc, xsd, dinv_ref, w1_ref, b1_ref,
             w2_ref, ys_ref):
        dinv = dinv_ref[...]
        p0q = (p0a, p0b, p0c, p0d)
        xsq = (xsa, xsb, xsc, xsd)
        p = jnp.concatenate(
            [(p0q[q][...] + xsq[q][...]) * dinv for q in range(NQ)], axis=1)
        h = jnp.dot(p, w1_ref[...], preferred_element_type=jnp.float32)
        h = jnp.maximum(h + b1_ref[...], 0.0)
        y = jnp.dot(h, w2_ref[...], preferred_element_type=jnp.float32)
        ys_ref[...] = y * dinv

    def qspec(q):
        return pl.BlockSpec((BM, QW), lambda i, q=q: (q * nb + i, 0))

    return pl.pallas_call(
        body,
        grid=(nb,),
        in_specs=[qspec(q) for q in range(NQ)] * 2 + [
            pl.BlockSpec((BM, 1), lambda i: (i, 0)),
            pl.BlockSpec((IN_CH, HID), lambda i: (0, 0)),
            pl.BlockSpec((1, HID), lambda i: (0, 0)),
            pl.BlockSpec((HID, 1), lambda i: (0, 0)),
        ],
        out_specs=pl.BlockSpec((BM, 1), lambda i: (i, 0)),
        out_shape=jax.ShapeDtypeStruct((NPAD, 1), jnp.float32),
    )(p0_st, p0_st, p0_st, p0_st, xs_st, xs_st, xs_st, xs_st,
      dinv, W1, b1, W2)


def _tc_final(oparts, ys, dinv, b2):
    BM = 1024
    nb = NPAD // BM

    def body(op0_ref, op1_ref, ys_ref, dinv_ref, b2_ref, out_ref):
        acc = op0_ref[...] + op1_ref[...] + ys_ref[...]
        out_ref[...] = dinv_ref[...] * acc + b2_ref[...]

    return pl.pallas_call(
        body,
        grid=(nb,),
        in_specs=[
            pl.BlockSpec((BM, 1), lambda i: (i, 0)),
            pl.BlockSpec((BM, 1), lambda i: (nb + i, 0)),
            pl.BlockSpec((BM, 1), lambda i: (i, 0)),
            pl.BlockSpec((BM, 1), lambda i: (i, 0)),
            pl.BlockSpec((1, 1), lambda i: (0, 0)),
        ],
        out_specs=pl.BlockSpec((BM, 1), lambda i: (i, 0)),
        out_shape=jax.ShapeDtypeStruct((NPAD, 1), jnp.float32),
    )(oparts, oparts, ys, dinv, b2)


def kernel(x, edge_index, W1, b1, W2, b2):
    n = x.shape[0]
    e = edge_index.shape[1]
    src = edge_index[0].astype(jnp.int32)
    dst = edge_index[1].astype(jnp.int32)
    pad_id = NPAD - 1
    src = jnp.pad(src, (0, EPAD - e), constant_values=pad_id)
    dst = jnp.pad(dst, (0, EPAD - e), constant_values=pad_id)
    src2d = src.reshape(EPAD // CHUNK, CHUNK)
    dst2d = dst.reshape(EPAD // CHUNK, CHUNK)
    x_pad = jnp.pad(x, ((0, NPAD - n), (0, 0)))

    degp = _sc_degree(dst2d).reshape(NC * NPAD, 1)
    dinv, xs_st = _tc_prep(degp, x_pad)
    p0_st = _sc_rowscatter(src, dst2d, xs_st)
    ys = _tc_mlp(p0_st, xs_st, dinv, W1, b1.reshape(1, HID), W2)
    op = _sc_scalar_scatter(src2d, dst2d, ys.reshape(NPAD))
    out = _tc_final(op.reshape(NC * NPAD, 1), ys, dinv, b2.reshape(1, 1))
    return out[:n]


# speedup vs baseline: 1.0611x; 1.0611x over previous
import functools

import jax
import jax.numpy as jnp
from jax import lax
from jax.experimental import pallas as pl
from jax.experimental.pallas import tpu as pltpu
from jax.experimental.pallas import tpu_sc as plsc

N_NODES = 10000
NPAD = 10240
IN_CH = 256
HID = 512
NC = 2
NS = 16
NW = NC * NS
CHUNK = 128
EPAD = 163840
NQ = 4
QW = IN_CH // NQ
NSEG = NPAD // NS


def _sc_mesh():
    return plsc.VectorSubcoreMesh(
        core_axis_name="c", subcore_axis_name="s", num_cores=NC, num_subcores=NS
    )


def _sc_degree(dst2d):
    rows_w = EPAD // NW // CHUNK

    @functools.partial(
        pl.kernel,
        out_type=jax.ShapeDtypeStruct((NC * NPAD,), jnp.float32),
        mesh=_sc_mesh(),
        scratch_types=[
            pltpu.VMEM((rows_w, CHUNK), jnp.int32),
            pltpu.VMEM((CHUNK,), jnp.float32),
            pltpu.VMEM((NSEG,), jnp.float32),
            pltpu.VMEM_SHARED((NPAD,), jnp.float32),
            pltpu.SemaphoreType.DMA,
        ],
    )
    def k(dst_hbm, out_hbm, idx_v, ones_v, stage_v, acc, sem):
        c = lax.axis_index("c")
        s = lax.axis_index("s")
        w = c * NS + s
        ones16 = jnp.ones((16,), jnp.float32)
        zero16 = jnp.zeros((16,), jnp.float32)
        for j in range(CHUNK // 16):
            ones_v[pl.ds(j * 16, 16)] = ones16

        def zfill(i, carry):
            stage_v[pl.ds(i * 16, 16)] = zero16
            return carry

        lax.fori_loop(0, NSEG // 16, zfill, 0)
        pltpu.sync_copy(stage_v, acc.at[pl.ds(s * NSEG, NSEG)])
        pltpu.sync_copy(dst_hbm.at[pl.ds(w * rows_w, rows_w)], idx_v)
        plsc.subcore_barrier()

        def fire(j, carry):
            pltpu.async_copy(ones_v, acc.at[idx_v.at[j]], sem, add=True)
            return carry

        lax.fori_loop(0, rows_w, fire, 0)

        def drain(j, carry):
            pltpu.make_async_copy(ones_v, acc.at[idx_v.at[0]], sem).wait()
            return carry

        lax.fori_loop(0, rows_w, drain, 0)
        plsc.subcore_barrier()
        pltpu.sync_copy(acc.at[pl.ds(s * NSEG, NSEG)], stage_v)
        pltpu.sync_copy(stage_v, out_hbm.at[pl.ds(c * NPAD + s * NSEG, NSEG)])

    return k(dst2d)


def _sc_rowscatter(src2d, dst2d, xs_st):
    rows_t = EPAD // NS // CHUNK

    @functools.partial(
        pl.kernel,
        out_type=jax.ShapeDtypeStruct((NQ * NPAD, QW), jnp.float32),
        mesh=_sc_mesh(),
        scratch_types=[
            pltpu.VMEM((rows_t, CHUNK), jnp.int32),
            pltpu.VMEM((rows_t, CHUNK), jnp.int32),
            pltpu.VMEM((CHUNK, QW), jnp.float32),
            pltpu.VMEM((CHUNK, QW), jnp.float32),
            pltpu.VMEM((CHUNK, QW), jnp.float32),
            pltpu.VMEM_SHARED((NPAD, QW), jnp.float32),
            pltpu.SemaphoreType.DMA,
            pltpu.SemaphoreType.DMA,
            pltpu.SemaphoreType.DMA,
        ],
        compiler_params=pltpu.CompilerParams(use_tc_tiling_on_sc=False),
    )
    def k(src_hbm, dst_hbm, xs_hbm, out_hbm, srcb, dstb, rowb, rowb1, rowb2,
          acc, gs0, gs1, gs2):
        c = lax.axis_index("c")
        s = lax.axis_index("s")
        zero16 = jnp.zeros((16,), jnp.float32)
        coff = jnp.broadcast_to(c * (2 * NPAD), (16,)).astype(jnp.int32)
        noff = jnp.broadcast_to(NPAD, (16,)).astype(jnp.int32)

        pltpu.sync_copy(src_hbm.at[pl.ds(s * rows_t, rows_t)], srcb)
        pltpu.sync_copy(dst_hbm.at[pl.ds(s * rows_t, rows_t)], dstb)

        def shift(i, carry):
            for j in range(CHUNK // 16):
                sl = pl.ds(j * 16, 16)
                srcb[i, sl] = srcb[i, sl] + coff
            return carry

        lax.fori_loop(0, rows_t, shift, 0)

        def shiftn(i, carry):
            for j in range(CHUNK // 16):
                sl = pl.ds(j * 16, 16)
                srcb[i, sl] = srcb[i, sl] + noff
            return carry

        pltpu.async_copy(xs_hbm.at[srcb.at[0]], rowb, gs0)
        pltpu.async_copy(xs_hbm.at[srcb.at[1]], rowb1, gs1)

        for q in range(2):
            if q == 0:
                bufs = (rowb, rowb1, rowb2)
                gsems = (gs0, gs1, gs2)
            else:
                bufs = (rowb1, rowb, rowb2)
                gsems = (gs1, gs0, gs2)
            zbuf = rowb2

            def zfill(i, carry):
                for j in range(QW // 16):
                    zbuf[i, pl.ds(j * 16, 16)] = zero16
                return carry

            lax.fori_loop(0, CHUNK, zfill, 0)

            def zacc(i, carry):
                pltpu.sync_copy(zbuf, acc.at[pl.ds(s * NSEG + i * CHUNK, CHUNK)])
                return carry

            lax.fori_loop(0, NSEG // CHUNK, zacc, 0)
            plsc.subcore_barrier()

            def gwait(b):
                pltpu.make_async_copy(
                    xs_hbm.at[srcb.at[0]], bufs[b], gsems[b]).wait()

            def body(jj, carry):
                for b in range(3):
                    j = jj * 3 + b
                    gwait(b)
                    jn = lax.rem(j + 2, rows_t)
                    pltpu.async_copy(xs_hbm.at[srcb.at[jn]], bufs[(b + 2) % 3],
                                     gsems[(b + 2) % 3])
                    pltpu.sync_copy(bufs[b], acc.at[dstb.at[j]], add=True)
                return carry

            lax.fori_loop(0, (rows_t - 2) // 3, body, 0)
            for j in (rows_t - 2, rows_t - 1):
                b = j % 3
                gwait(b)
                pltpu.sync_copy(bufs[b], acc.at[dstb.at[j]], add=True)
            plsc.subcore_barrier()

            if q == 0:
                lax.fori_loop(0, rows_t, shiftn, 0)
                pltpu.async_copy(xs_hbm.at[srcb.at[0]], rowb1, gs1)
                pltpu.async_copy(xs_hbm.at[srcb.at[1]], rowb, gs0)

            def drain(i, carry):
                sl = pl.ds(s * NSEG + i * CHUNK, CHUNK)
                pltpu.sync_copy(acc.at[sl], zbuf)
                pltpu.sync_copy(
                    zbuf,
                    out_hbm.at[pl.ds((2 * c + q) * NPAD + s * NSEG + i * CHUNK, CHUNK)],
                )
                return carry

            lax.fori_loop(0, NSEG // CHUNK, drain, 0)

    return k(src2d, dst2d, xs_st)


def _sc_scalar_scatter(src2d, dst2d, ys):
    rows_w = EPAD // NW // CHUNK

    @functools.partial(
        pl.kernel,
        out_type=jax.ShapeDtypeStruct((NC * NPAD,), jnp.float32),
        mesh=_sc_mesh(),
        scratch_types=[
            pltpu.VMEM((rows_w, CHUNK), jnp.int32),
            pltpu.VMEM((rows_w, CHUNK), jnp.int32),
            pltpu.VMEM((rows_w, CHUNK), jnp.float32),
            pltpu.VMEM((NSEG,), jnp.float32),
            pltpu.VMEM_SHARED((NPAD,), jnp.float32),
            pltpu.SemaphoreType.DMA,
        ],
    )
    def k(src_hbm, dst_hbm, ys_hbm, out_hbm, srcb, dstb, vals, stage_v, acc, sem):
        c = lax.axis_index("c")
        s = lax.axis_index("s")
        w = c * NS + s
        zero16 = jnp.zeros((16,), jnp.float32)

        def zfill(i, carry):
            stage_v[pl.ds(i * 16, 16)] = zero16
            return carry

        lax.fori_loop(0, NSEG // 16, zfill, 0)
        pltpu.sync_copy(stage_v, acc.at[pl.ds(s * NSEG, NSEG)])
        pltpu.sync_copy(src_hbm.at[pl.ds(w * rows_w, rows_w)], srcb)
        pltpu.sync_copy(dst_hbm.at[pl.ds(w * rows_w, rows_w)], dstb)

        def gfire(j, carry):
            pltpu.async_copy(ys_hbm.at[srcb.at[j]], vals.at[j], sem)
            return carry

        lax.fori_loop(0, rows_w, gfire, 0)

        def gdrain(j, carry):
            pltpu.make_async_copy(ys_hbm.at[srcb.at[0]], vals.at[0], sem).wait()
            return carry

        lax.fori_loop(0, rows_w, gdrain, 0)
        plsc.subcore_barrier()

        def sfire(j, carry):
            pltpu.async_copy(vals.at[j], acc.at[dstb.at[j]], sem, add=True)
            return carry

        lax.fori_loop(0, rows_w, sfire, 0)

        def sdrain(j, carry):
            pltpu.make_async_copy(vals.at[0], acc.at[dstb.at[0]], sem).wait()
            return carry

        lax.fori_loop(0, rows_w, sdrain, 0)
        plsc.subcore_barrier()
        pltpu.sync_copy(acc.at[pl.ds(s * NSEG, NSEG)], stage_v)
        pltpu.sync_copy(stage_v, out_hbm.at[pl.ds(c * NPAD + s * NSEG, NSEG)])

    return k(src2d, dst2d, ys)


def _tc_prep(degp, x_pad):
    BM = 1024
    nb = NPAD // BM

    def body(dp0_ref, dp1_ref, x_ref, dinv_ref, xs_ref):
        deg = 1.0 + dp0_ref[...] + dp1_ref[...]
        dinv = lax.rsqrt(deg)
        dinv_ref[...] = dinv
        for q in range(NQ):
            xs_ref[q] = x_ref[:, pl.ds(q * QW, QW)] * dinv

    dinv, xs4 = pl.pallas_call(
        body,
        grid=(nb,),
        in_specs=[
            pl.BlockSpec((BM, 1), lambda i: (i, 0)),
            pl.BlockSpec((BM, 1), lambda i: (nb + i, 0)),
            pl.BlockSpec((BM, IN_CH), lambda i: (i, 0)),
        ],
        out_specs=[
            pl.BlockSpec((BM, 1), lambda i: (i, 0)),
            pl.BlockSpec((NQ, BM, QW), lambda i: (0, i, 0)),
        ],
        out_shape=[
            jax.ShapeDtypeStruct((NPAD, 1), jnp.float32),
            jax.ShapeDtypeStruct((NQ, NPAD, QW), jnp.float32),
        ],
    )(degp, degp, x_pad)
    return dinv, xs4.reshape(NQ * NPAD, QW)


def _tc_mlp(p0_st, xs_st, dinv, W1, b1, W2):
    BM = 512
    nb = NPAD // BM

    def body(p0a, p0b, p0c, p0d, xsa, xsb, xsc, xsd, dinv_ref, w1_ref, b1_ref,
             w2_ref, ys_ref):
        dinv = dinv_ref[...]
        p0q = (p0a, p0b, p0c, p0d)
        xsq = (xsa, xsb, xsc, xsd)
        p = jnp.concatenate(
            [(p0q[q][...] + xsq[q][...]) * dinv for q in range(NQ)], axis=1)
        h = jnp.dot(p, w1_ref[...], preferred_element_type=jnp.float32)
        h = jnp.maximum(h + b1_ref[...], 0.0)
        y = jnp.dot(h, w2_ref[...], preferred_element_type=jnp.float32)
        ys_ref[...] = y * dinv

    def qspec(q):
        return pl.BlockSpec((BM, QW), lambda i, q=q: (q * nb + i, 0))

    return pl.pallas_call(
        body,
        grid=(nb,),
        in_specs=[qspec(q) for q in range(NQ)] * 2 + [
            pl.BlockSpec((BM, 1), lambda i: (i, 0)),
            pl.BlockSpec((IN_CH, HID), lambda i: (0, 0)),
            pl.BlockSpec((1, HID), lambda i: (0, 0)),
            pl.BlockSpec((HID, 1), lambda i: (0, 0)),
        ],
        out_specs=pl.BlockSpec((BM, 1), lambda i: (i, 0)),
        out_shape=jax.ShapeDtypeStruct((NPAD, 1), jnp.float32),
    )(p0_st, p0_st, p0_st, p0_st, xs_st, xs_st, xs_st, xs_st,
      dinv, W1, b1, W2)


def _tc_final(oparts, ys, dinv, b2):
    BM = 1024
    nb = NPAD // BM

    def body(op0_ref, op1_ref, ys_ref, dinv_ref, b2_ref, out_ref):
        acc = op0_ref[...] + op1_ref[...] + ys_ref[...]
        out_ref[...] = dinv_ref[...] * acc + b2_ref[...]

    return pl.pallas_call(
        body,
        grid=(nb,),
        in_specs=[
            pl.BlockSpec((BM, 1), lambda i: (i, 0)),
            pl.BlockSpec((BM, 1), lambda i: (nb + i, 0)),
            pl.BlockSpec((BM, 1), lambda i: (i, 0)),
            pl.BlockSpec((BM, 1), lambda i: (i, 0)),
            pl.BlockSpec((1, 1), lambda i: (0, 0)),
        ],
        out_specs=pl.BlockSpec((BM, 1), lambda i: (i, 0)),
        out_shape=jax.ShapeDtypeStruct((NPAD, 1), jnp.float32),
    )(oparts, oparts, ys, dinv, b2)


def kernel(x, edge_index, W1, b1, W2, b2):
    n = x.shape[0]
    e = edge_index.shape[1]
    src = edge_index[0].astype(jnp.int32)
    dst = edge_index[1].astype(jnp.int32)
    pad_id = NPAD - 1
    src = jnp.pad(src, (0, EPAD - e), constant_values=pad_id)
    dst = jnp.pad(dst, (0, EPAD - e), constant_values=pad_id)
    src2d = src.reshape(EPAD // CHUNK, CHUNK)
    dst2d = dst.reshape(EPAD // CHUNK, CHUNK)
    x_pad = jnp.pad(x, ((0, NPAD - n), (0, 0)))

    degp = _sc_degree(dst2d).reshape(NC * NPAD, 1)
    dinv, xs_st = _tc_prep(degp, x_pad)
    p0_st = _sc_rowscatter(src2d, dst2d, xs_st)
    ys = _tc_mlp(p0_st, xs_st, dinv, W1, b1.reshape(1, HID), W2)
    op = _sc_scalar_scatter(src2d, dst2d, ys.reshape(NPAD))
    out = _tc_final(op.reshape(NC * NPAD, 1), ys, dinv, b2.reshape(1, 1))
    return out[:n]


# speedup vs baseline: 1.0763x; 1.0143x over previous
import functools

import jax
import jax.numpy as jnp
from jax import lax
from jax.experimental import pallas as pl
from jax.experimental.pallas import tpu as pltpu
from jax.experimental.pallas import tpu_sc as plsc

N_NODES = 10000
NPAD = 10240
IN_CH = 256
HID = 512
NC = 2
NS = 16
NW = NC * NS
CHUNK = 128
EPAD = 163840
NQ = 4
QW = IN_CH // NQ
NSEG = NPAD // NS


def _sc_mesh():
    return plsc.VectorSubcoreMesh(
        core_axis_name="c", subcore_axis_name="s", num_cores=NC, num_subcores=NS
    )


def _sc_degree(dst2d):
    rows_w = EPAD // NW // CHUNK

    @functools.partial(
        pl.kernel,
        out_type=jax.ShapeDtypeStruct((NC * NPAD,), jnp.float32),
        mesh=_sc_mesh(),
        scratch_types=[
            pltpu.VMEM((rows_w, CHUNK), jnp.int32),
            pltpu.VMEM((CHUNK,), jnp.float32),
            pltpu.VMEM((NSEG,), jnp.float32),
            pltpu.VMEM_SHARED((NPAD,), jnp.float32),
            pltpu.SemaphoreType.DMA,
        ],
    )
    def k(dst_hbm, out_hbm, idx_v, ones_v, stage_v, acc, sem):
        c = lax.axis_index("c")
        s = lax.axis_index("s")
        w = c * NS + s
        ones16 = jnp.ones((16,), jnp.float32)
        zero16 = jnp.zeros((16,), jnp.float32)
        for j in range(CHUNK // 16):
            ones_v[pl.ds(j * 16, 16)] = ones16

        def zfill(i, carry):
            stage_v[pl.ds(i * 16, 16)] = zero16
            return carry

        lax.fori_loop(0, NSEG // 16, zfill, 0)
        pltpu.sync_copy(stage_v, acc.at[pl.ds(s * NSEG, NSEG)])
        pltpu.sync_copy(dst_hbm.at[pl.ds(w * rows_w, rows_w)], idx_v)
        plsc.subcore_barrier()

        def fire(j, carry):
            pltpu.async_copy(ones_v, acc.at[idx_v.at[j]], sem, add=True)
            return carry

        lax.fori_loop(0, rows_w, fire, 0)

        def drain(j, carry):
            pltpu.make_async_copy(ones_v, acc.at[idx_v.at[0]], sem).wait()
            return carry

        lax.fori_loop(0, rows_w, drain, 0)
        plsc.subcore_barrier()
        pltpu.sync_copy(acc.at[pl.ds(s * NSEG, NSEG)], stage_v)
        pltpu.sync_copy(stage_v, out_hbm.at[pl.ds(c * NPAD + s * NSEG, NSEG)])

    return k(dst2d)


def _sc_rowscatter(src2d, dst2d, xs_st):
    rows_t = EPAD // NS // CHUNK

    @functools.partial(
        pl.kernel,
        out_type=jax.ShapeDtypeStruct((NQ * NPAD, QW), jnp.float32),
        mesh=_sc_mesh(),
        scratch_types=[
            pltpu.VMEM((rows_t, CHUNK), jnp.int32),
            pltpu.VMEM((rows_t, CHUNK), jnp.int32),
            pltpu.VMEM((CHUNK, QW), jnp.float32),
            pltpu.VMEM((CHUNK, QW), jnp.float32),
            pltpu.VMEM((CHUNK, QW), jnp.float32),
            pltpu.VMEM_SHARED((NPAD, QW), jnp.float32),
            pltpu.SemaphoreType.DMA,
            pltpu.SemaphoreType.DMA,
            pltpu.SemaphoreType.DMA,
        ],
        compiler_params=pltpu.CompilerParams(use_tc_tiling_on_sc=False),
    )
    def k(src_hbm, dst_hbm, xs_hbm, out_hbm, srcb, dstb, rowb, rowb1, rowb2,
          acc, gs0, gs1, gs2):
        c = lax.axis_index("c")
        s = lax.axis_index("s")
        zero16 = jnp.zeros((16,), jnp.float32)
        coff = jnp.broadcast_to(c * (2 * NPAD), (16,)).astype(jnp.int32)
        noff = jnp.broadcast_to(NPAD, (16,)).astype(jnp.int32)

        pltpu.sync_copy(src_hbm.at[pl.ds(s * rows_t, rows_t)], srcb)
        pltpu.sync_copy(dst_hbm.at[pl.ds(s * rows_t, rows_t)], dstb)

        def shift(i, carry):
            for j in range(CHUNK // 16):
                sl = pl.ds(j * 16, 16)
                srcb[i, sl] = srcb[i, sl] + coff
            return carry

        lax.fori_loop(0, rows_t, shift, 0)

        def shiftn(i, carry):
            for j in range(CHUNK // 16):
                sl = pl.ds(j * 16, 16)
                srcb[i, sl] = srcb[i, sl] + noff
            return carry

        pltpu.async_copy(xs_hbm.at[srcb.at[0]], rowb, gs0)
        pltpu.async_copy(xs_hbm.at[srcb.at[1]], rowb1, gs1)

        for q in range(2):
            if q == 0:
                bufs = (rowb, rowb1, rowb2)
                gsems = (gs0, gs1, gs2)
            else:
                bufs = (rowb1, rowb, rowb2)
                gsems = (gs1, gs0, gs2)
            zbuf = rowb2

            def zfill(i, carry):
                for j in range(QW // 16):
                    zbuf[i, pl.ds(j * 16, 16)] = zero16
                return carry

            lax.fori_loop(0, CHUNK, zfill, 0)

            def zacc(i, carry):
                pltpu.sync_copy(zbuf, acc.at[pl.ds(s * NSEG + i * CHUNK, CHUNK)])
                return carry

            lax.fori_loop(0, NSEG // CHUNK, zacc, 0)
            plsc.subcore_barrier()

            def gwait(b):
                pltpu.make_async_copy(
                    xs_hbm.at[srcb.at[0]], bufs[b], gsems[b]).wait()

            def body(jj, carry):
                for b in range(3):
                    j = jj * 3 + b
                    gwait(b)
                    jn = lax.rem(j + 2, rows_t)
                    pltpu.async_copy(xs_hbm.at[srcb.at[jn]], bufs[(b + 2) % 3],
                                     gsems[(b + 2) % 3])
                    pltpu.sync_copy(bufs[b], acc.at[dstb.at[j]], add=True)
                return carry

            lax.fori_loop(0, (rows_t - 2) // 3, body, 0)
            for j in (rows_t - 2, rows_t - 1):
                b = j % 3
                gwait(b)
                pltpu.sync_copy(bufs[b], acc.at[dstb.at[j]], add=True)
            plsc.subcore_barrier()

            if q == 0:
                lax.fori_loop(0, rows_t, shiftn, 0)
                pltpu.async_copy(xs_hbm.at[srcb.at[0]], rowb1, gs1)
                pltpu.async_copy(xs_hbm.at[srcb.at[1]], rowb, gs0)

            def drain(i, carry):
                sl = pl.ds(s * NSEG + i * CHUNK, CHUNK)
                pltpu.sync_copy(acc.at[sl], zbuf)
                pltpu.sync_copy(
                    zbuf,
                    out_hbm.at[pl.ds((2 * c + q) * NPAD + s * NSEG + i * CHUNK, CHUNK)],
                )
                return carry

            lax.fori_loop(0, NSEG // CHUNK, drain, 0)

    return k(src2d, dst2d, xs_st)


def _sc_scalar_scatter(src2d, dst2d, ys):
    rows_w = EPAD // NW // CHUNK

    @functools.partial(
        pl.kernel,
        out_type=jax.ShapeDtypeStruct((NC * NPAD,), jnp.float32),
        mesh=_sc_mesh(),
        scratch_types=[
            pltpu.VMEM((rows_w, CHUNK), jnp.int32),
            pltpu.VMEM((rows_w, CHUNK), jnp.int32),
            pltpu.VMEM((rows_w, CHUNK), jnp.float32),
            pltpu.VMEM((NSEG,), jnp.float32),
            pltpu.VMEM_SHARED((NPAD,), jnp.float32),
            pltpu.SemaphoreType.DMA,
        ],
    )
    def k(src_hbm, dst_hbm, ys_hbm, out_hbm, srcb, dstb, vals, stage_v, acc, sem):
        c = lax.axis_index("c")
        s = lax.axis_index("s")
        w = c * NS + s
        zero16 = jnp.zeros((16,), jnp.float32)

        def zfill(i, carry):
            stage_v[pl.ds(i * 16, 16)] = zero16
            return carry

        lax.fori_loop(0, NSEG // 16, zfill, 0)
        pltpu.sync_copy(stage_v, acc.at[pl.ds(s * NSEG, NSEG)])
        pltpu.sync_copy(src_hbm.at[pl.ds(w * rows_w, rows_w)], srcb)
        pltpu.sync_copy(dst_hbm.at[pl.ds(w * rows_w, rows_w)], dstb)

        def gfire(j, carry):
            pltpu.async_copy(ys_hbm.at[srcb.at[j]], vals.at[j], sem)
            return carry

        lax.fori_loop(0, rows_w, gfire, 0)

        def gdrain(j, carry):
            pltpu.make_async_copy(ys_hbm.at[srcb.at[0]], vals.at[0], sem).wait()
            return carry

        lax.fori_loop(0, rows_w, gdrain, 0)
        plsc.subcore_barrier()

        def sfire(j, carry):
            pltpu.async_copy(vals.at[j], acc.at[dstb.at[j]], sem, add=True)
            return carry

        lax.fori_loop(0, rows_w, sfire, 0)

        def sdrain(j, carry):
            pltpu.make_async_copy(vals.at[0], acc.at[dstb.at[0]], sem).wait()
            return carry

        lax.fori_loop(0, rows_w, sdrain, 0)
        plsc.subcore_barrier()
        pltpu.sync_copy(acc.at[pl.ds(s * NSEG, NSEG)], stage_v)
        pltpu.sync_copy(stage_v, out_hbm.at[pl.ds(c * NPAD + s * NSEG, NSEG)])

    return k(src2d, dst2d, ys)


def _tc_prep(degp, x_pad):
    BM = 1024
    nb = NPAD // BM

    def body(dp0_ref, dp1_ref, x_ref, dinv_ref, xs_ref):
        deg = 1.0 + dp0_ref[...] + dp1_ref[...]
        dinv = lax.rsqrt(deg)
        dinv_ref[...] = dinv
        for q in range(NQ):
            xs_ref[q] = x_ref[:, pl.ds(q * QW, QW)] * dinv

    dinv, xs4 = pl.pallas_call(
        body,
        grid=(nb,),
        in_specs=[
            pl.BlockSpec((BM, 1), lambda i: (i, 0)),
            pl.BlockSpec((BM, 1), lambda i: (nb + i, 0)),
            pl.BlockSpec((BM, IN_CH), lambda i: (i, 0)),
        ],
        out_specs=[
            pl.BlockSpec((BM, 1), lambda i: (i, 0)),
            pl.BlockSpec((NQ, BM, QW), lambda i: (0, i, 0)),
        ],
        out_shape=[
            jax.ShapeDtypeStruct((NPAD, 1), jnp.float32),
            jax.ShapeDtypeStruct((NQ, NPAD, QW), jnp.float32),
        ],
    )(degp, degp, x_pad)
    return dinv, xs4.reshape(NQ * NPAD, QW)


def _tc_mlp(p0_st, xs_st, dinv, W1, b1, W2):
    BM = 1024
    nb = NPAD // BM

    def body(p0a, p0b, p0c, p0d, xsa, xsb, xsc, xsd, dinv_ref, w1_ref, b1_ref,
             w2_ref, ys_ref):
        dinv = dinv_ref[...]
        p0q = (p0a, p0b, p0c, p0d)
        xsq = (xsa, xsb, xsc, xsd)
        p = jnp.concatenate(
            [(p0q[q][...] + xsq[q][...]) * dinv for q in range(NQ)], axis=1)
        h = jnp.dot(p, w1_ref[...], preferred_element_type=jnp.float32)
        h = jnp.maximum(h + b1_ref[...], 0.0)
        y = jnp.dot(h, w2_ref[...], preferred_element_type=jnp.float32)
        ys_ref[...] = y * dinv

    def qspec(q):
        return pl.BlockSpec((BM, QW), lambda i, q=q: (q * nb + i, 0))

    return pl.pallas_call(
        body,
        grid=(nb,),
        in_specs=[qspec(q) for q in range(NQ)] * 2 + [
            pl.BlockSpec((BM, 1), lambda i: (i, 0)),
            pl.BlockSpec((IN_CH, HID), lambda i: (0, 0)),
            pl.BlockSpec((1, HID), lambda i: (0, 0)),
            pl.BlockSpec((HID, 1), lambda i: (0, 0)),
        ],
        out_specs=pl.BlockSpec((BM, 1), lambda i: (i, 0)),
        out_shape=jax.ShapeDtypeStruct((NPAD, 1), jnp.float32),
    )(p0_st, p0_st, p0_st, p0_st, xs_st, xs_st, xs_st, xs_st,
      dinv, W1, b1, W2)


def _tc_final(oparts, ys, dinv, b2):
    BM = 1024
    nb = NPAD // BM

    def body(op0_ref, op1_ref, ys_ref, dinv_ref, b2_ref, out_ref):
        acc = op0_ref[...] + op1_ref[...] + ys_ref[...]
        out_ref[...] = dinv_ref[...] * acc + b2_ref[...]

    return pl.pallas_call(
        body,
        grid=(nb,),
        in_specs=[
            pl.BlockSpec((BM, 1), lambda i: (i, 0)),
            pl.BlockSpec((BM, 1), lambda i: (nb + i, 0)),
            pl.BlockSpec((BM, 1), lambda i: (i, 0)),
            pl.BlockSpec((BM, 1), lambda i: (i, 0)),
            pl.BlockSpec((1, 1), lambda i: (0, 0)),
        ],
        out_specs=pl.BlockSpec((BM, 1), lambda i: (i, 0)),
        out_shape=jax.ShapeDtypeStruct((NPAD, 1), jnp.float32),
    )(oparts, oparts, ys, dinv, b2)


def kernel(x, edge_index, W1, b1, W2, b2):
    n = x.shape[0]
    e = edge_index.shape[1]
    src = edge_index[0].astype(jnp.int32)
    dst = edge_index[1].astype(jnp.int32)
    pad_id = NPAD - 1
    src = jnp.pad(src, (0, EPAD - e), constant_values=pad_id)
    dst = jnp.pad(dst, (0, EPAD - e), constant_values=pad_id)
    src2d = src.reshape(EPAD // CHUNK, CHUNK)
    dst2d = dst.reshape(EPAD // CHUNK, CHUNK)
    x_pad = jnp.pad(x, ((0, NPAD - n), (0, 0)))

    degp = _sc_degree(dst2d).reshape(NC * NPAD, 1)
    dinv, xs_st = _tc_prep(degp, x_pad)
    p0_st = _sc_rowscatter(src2d, dst2d, xs_st)
    ys = _tc_mlp(p0_st, xs_st, dinv, W1, b1.reshape(1, HID), W2)
    op = _sc_scalar_scatter(src2d, dst2d, ys.reshape(NPAD))
    out = _tc_final(op.reshape(NC * NPAD, 1), ys, dinv, b2.reshape(1, 1))
    return out[:n]


# speedup vs baseline: 1.0793x; 1.0028x over previous
import functools

import jax
import jax.numpy as jnp
from jax import lax
from jax.experimental import pallas as pl
from jax.experimental.pallas import tpu as pltpu
from jax.experimental.pallas import tpu_sc as plsc

N_NODES = 10000
NPAD = 10240
IN_CH = 256
HID = 512
NC = 2
NS = 16
NW = NC * NS
CHUNK = 128
EPAD = 163840
NQ = 4
QW = IN_CH // NQ
NSEG = NPAD // NS


def _sc_mesh():
    return plsc.VectorSubcoreMesh(
        core_axis_name="c", subcore_axis_name="s", num_cores=NC, num_subcores=NS
    )


def _sc_degree(dst2d):
    rows_w = EPAD // NW // CHUNK

    @functools.partial(
        pl.kernel,
        out_type=jax.ShapeDtypeStruct((NC * NPAD,), jnp.float32),
        mesh=_sc_mesh(),
        scratch_types=[
            pltpu.VMEM((rows_w, CHUNK), jnp.int32),
            pltpu.VMEM((CHUNK,), jnp.float32),
            pltpu.VMEM((NSEG,), jnp.float32),
            pltpu.VMEM_SHARED((NPAD,), jnp.float32),
            pltpu.SemaphoreType.DMA,
        ],
    )
    def k(dst_hbm, out_hbm, idx_v, ones_v, stage_v, acc, sem):
        c = lax.axis_index("c")
        s = lax.axis_index("s")
        w = c * NS + s
        ones16 = jnp.ones((16,), jnp.float32)
        zero16 = jnp.zeros((16,), jnp.float32)
        for j in range(CHUNK // 16):
            ones_v[pl.ds(j * 16, 16)] = ones16

        def zfill(i, carry):
            stage_v[pl.ds(i * 16, 16)] = zero16
            return carry

        lax.fori_loop(0, NSEG // 16, zfill, 0)
        pltpu.sync_copy(stage_v, acc.at[pl.ds(s * NSEG, NSEG)])
        pltpu.sync_copy(dst_hbm.at[pl.ds(w * rows_w, rows_w)], idx_v)
        plsc.subcore_barrier()

        def fire(j, carry):
            pltpu.async_copy(ones_v, acc.at[idx_v.at[j]], sem, add=True)
            return carry

        lax.fori_loop(0, rows_w, fire, 0)

        def drain(j, carry):
            pltpu.make_async_copy(ones_v, acc.at[idx_v.at[0]], sem).wait()
            return carry

        lax.fori_loop(0, rows_w, drain, 0)
        plsc.subcore_barrier()
        pltpu.sync_copy(acc.at[pl.ds(s * NSEG, NSEG)], stage_v)
        pltpu.sync_copy(stage_v, out_hbm.at[pl.ds(c * NPAD + s * NSEG, NSEG)])

    return k(dst2d)


def _sc_rowscatter(src2d, dst2d, xs_st):
    rows_t = EPAD // NS // CHUNK

    @functools.partial(
        pl.kernel,
        out_type=jax.ShapeDtypeStruct((NQ * NPAD, QW), jnp.float32),
        mesh=_sc_mesh(),
        scratch_types=[
            pltpu.VMEM((rows_t, CHUNK), jnp.int32),
            pltpu.VMEM((rows_t, CHUNK), jnp.int32),
            pltpu.VMEM((CHUNK, QW), jnp.float32),
            pltpu.VMEM((CHUNK, QW), jnp.float32),
            pltpu.VMEM((CHUNK, QW), jnp.float32),
            pltpu.VMEM_SHARED((NPAD, QW), jnp.float32),
            pltpu.SemaphoreType.DMA,
            pltpu.SemaphoreType.DMA,
            pltpu.SemaphoreType.DMA,
        ],
        compiler_params=pltpu.CompilerParams(use_tc_tiling_on_sc=False),
    )
    def k(src_hbm, dst_hbm, xs_hbm, out_hbm, srcb, dstb, rowb, rowb1, rowb2,
          acc, gs0, gs1, gs2):
        c = lax.axis_index("c")
        s = lax.axis_index("s")
        zero16 = jnp.zeros((16,), jnp.float32)
        coff = jnp.broadcast_to(c * (2 * NPAD), (16,)).astype(jnp.int32)
        noff = jnp.broadcast_to(NPAD, (16,)).astype(jnp.int32)

        pltpu.sync_copy(src_hbm.at[pl.ds(s * rows_t, rows_t)], srcb)
        pltpu.sync_copy(dst_hbm.at[pl.ds(s * rows_t, rows_t)], dstb)

        def shift(i, carry):
            for j in range(CHUNK // 16):
                sl = pl.ds(j * 16, 16)
                srcb[i, sl] = srcb[i, sl] + coff
            return carry

        lax.fori_loop(0, rows_t, shift, 0)

        def shiftn(i, carry):
            for j in range(CHUNK // 16):
                sl = pl.ds(j * 16, 16)
                srcb[i, sl] = srcb[i, sl] + noff
            return carry

        pltpu.async_copy(xs_hbm.at[srcb.at[0]], rowb, gs0)
        pltpu.async_copy(xs_hbm.at[srcb.at[1]], rowb1, gs1)

        for q in range(2):
            if q == 0:
                bufs = (rowb, rowb1, rowb2)
                gsems = (gs0, gs1, gs2)
            else:
                bufs = (rowb1, rowb, rowb2)
                gsems = (gs1, gs0, gs2)
            zbuf = rowb2

            def zfill(i, carry):
                for j in range(QW // 16):
                    zbuf[i, pl.ds(j * 16, 16)] = zero16
                return carry

            lax.fori_loop(0, CHUNK, zfill, 0)

            def zacc(i, carry):
                pltpu.sync_copy(zbuf, acc.at[pl.ds(s * NSEG + i * CHUNK, CHUNK)])
                return carry

            lax.fori_loop(0, NSEG // CHUNK, zacc, 0)
            plsc.subcore_barrier()

            def gwait(b):
                pltpu.make_async_copy(
                    xs_hbm.at[srcb.at[0]], bufs[b], gsems[b]).wait()

            def body(jj, carry):
                for b in range(3):
                    j = jj * 3 + b
                    gwait(b)
                    jn = lax.rem(j + 2, rows_t)
                    pltpu.async_copy(xs_hbm.at[srcb.at[jn]], bufs[(b + 2) % 3],
                                     gsems[(b + 2) % 3])
                    pltpu.sync_copy(bufs[b], acc.at[dstb.at[j]], add=True)
                return carry

            lax.fori_loop(0, (rows_t - 2) // 3, body, 0)
            for j in (rows_t - 2, rows_t - 1):
                b = j % 3
                gwait(b)
                pltpu.sync_copy(bufs[b], acc.at[dstb.at[j]], add=True)
            plsc.subcore_barrier()

            if q == 0:
                lax.fori_loop(0, rows_t, shiftn, 0)
                pltpu.async_copy(xs_hbm.at[srcb.at[0]], rowb1, gs1)
                pltpu.async_copy(xs_hbm.at[srcb.at[1]], rowb, gs0)

            def drain(i, carry):
                sl = pl.ds(s * NSEG + i * CHUNK, CHUNK)
                pltpu.sync_copy(acc.at[sl], zbuf)
                pltpu.sync_copy(
                    zbuf,
                    out_hbm.at[pl.ds((2 * c + q) * NPAD + s * NSEG + i * CHUNK, CHUNK)],
                )
                return carry

            lax.fori_loop(0, NSEG // CHUNK, drain, 0)

    return k(src2d, dst2d, xs_st)


def _sc_scalar_scatter(src2d, dst2d, ys):
    rows_w = EPAD // NW // CHUNK

    @functools.partial(
        pl.kernel,
        out_type=jax.ShapeDtypeStruct((NC * NPAD,), jnp.float32),
        mesh=_sc_mesh(),
        scratch_types=[
            pltpu.VMEM((rows_w, CHUNK), jnp.int32),
            pltpu.VMEM((rows_w, CHUNK), jnp.int32),
            pltpu.VMEM((rows_w, CHUNK), jnp.float32),
            pltpu.VMEM((NSEG,), jnp.float32),
            pltpu.VMEM_SHARED((NPAD,), jnp.float32),
            pltpu.SemaphoreType.DMA,
        ],
    )
    def k(src_hbm, dst_hbm, ys_hbm, out_hbm, srcb, dstb, vals, stage_v, acc, sem):
        c = lax.axis_index("c")
        s = lax.axis_index("s")
        w = c * NS + s
        zero16 = jnp.zeros((16,), jnp.float32)

        def zfill(i, carry):
            stage_v[pl.ds(i * 16, 16)] = zero16
            return carry

        lax.fori_loop(0, NSEG // 16, zfill, 0)
        pltpu.sync_copy(stage_v, acc.at[pl.ds(s * NSEG, NSEG)])
        pltpu.sync_copy(src_hbm.at[pl.ds(w * rows_w, rows_w)], srcb)
        pltpu.sync_copy(dst_hbm.at[pl.ds(w * rows_w, rows_w)], dstb)

        def gfire(j, carry):
            pltpu.async_copy(ys_hbm.at[srcb.at[j]], vals.at[j], sem)
            return carry

        lax.fori_loop(0, rows_w, gfire, 0)

        def gdrain(j, carry):
            pltpu.make_async_copy(ys_hbm.at[srcb.at[0]], vals.at[0], sem).wait()
            return carry

        lax.fori_loop(0, rows_w, gdrain, 0)
        plsc.subcore_barrier()

        def sfire(j, carry):
            pltpu.async_copy(vals.at[j], acc.at[dstb.at[j]], sem, add=True)
            return carry

        lax.fori_loop(0, rows_w, sfire, 0)

        def sdrain(j, carry):
            pltpu.make_async_copy(vals.at[0], acc.at[dstb.at[0]], sem).wait()
            return carry

        lax.fori_loop(0, rows_w, sdrain, 0)
        plsc.subcore_barrier()
        pltpu.sync_copy(acc.at[pl.ds(s * NSEG, NSEG)], stage_v)
        pltpu.sync_copy(stage_v, out_hbm.at[pl.ds(c * NPAD + s * NSEG, NSEG)])

    return k(src2d, dst2d, ys)


def _tc_prep(degp, x_pad):
    BM = 2048
    nb = NPAD // BM

    def body(dp0_ref, dp1_ref, x_ref, dinv_ref, xs_ref):
        deg = 1.0 + dp0_ref[...] + dp1_ref[...]
        dinv = lax.rsqrt(deg)
        dinv_ref[...] = dinv
        for q in range(NQ):
            xs_ref[q] = x_ref[:, pl.ds(q * QW, QW)] * dinv

    dinv, xs4 = pl.pallas_call(
        body,
        grid=(nb,),
        in_specs=[
            pl.BlockSpec((BM, 1), lambda i: (i, 0)),
            pl.BlockSpec((BM, 1), lambda i: (nb + i, 0)),
            pl.BlockSpec((BM, IN_CH), lambda i: (i, 0)),
        ],
        out_specs=[
            pl.BlockSpec((BM, 1), lambda i: (i, 0)),
            pl.BlockSpec((NQ, BM, QW), lambda i: (0, i, 0)),
        ],
        out_shape=[
            jax.ShapeDtypeStruct((NPAD, 1), jnp.float32),
            jax.ShapeDtypeStruct((NQ, NPAD, QW), jnp.float32),
        ],
    )(degp, degp, x_pad)
    return dinv, xs4.reshape(NQ * NPAD, QW)


def _tc_mlp(p0_st, xs_st, dinv, W1, b1, W2):
    BM = 1024
    nb = NPAD // BM

    def body(p0a, p0b, p0c, p0d, xsa, xsb, xsc, xsd, dinv_ref, w1_ref, b1_ref,
             w2_ref, ys_ref):
        dinv = dinv_ref[...]
        p0q = (p0a, p0b, p0c, p0d)
        xsq = (xsa, xsb, xsc, xsd)
        p = jnp.concatenate(
            [(p0q[q][...] + xsq[q][...]) * dinv for q in range(NQ)], axis=1)
        h = jnp.dot(p, w1_ref[...], preferred_element_type=jnp.float32)
        h = jnp.maximum(h + b1_ref[...], 0.0)
        y = jnp.dot(h, w2_ref[...], preferred_element_type=jnp.float32)
        ys_ref[...] = y * dinv

    def qspec(q):
        return pl.BlockSpec((BM, QW), lambda i, q=q: (q * nb + i, 0))

    return pl.pallas_call(
        body,
        grid=(nb,),
        in_specs=[qspec(q) for q in range(NQ)] * 2 + [
            pl.BlockSpec((BM, 1), lambda i: (i, 0)),
            pl.BlockSpec((IN_CH, HID), lambda i: (0, 0)),
            pl.BlockSpec((1, HID), lambda i: (0, 0)),
            pl.BlockSpec((HID, 1), lambda i: (0, 0)),
        ],
        out_specs=pl.BlockSpec((BM, 1), lambda i: (i, 0)),
        out_shape=jax.ShapeDtypeStruct((NPAD, 1), jnp.float32),
    )(p0_st, p0_st, p0_st, p0_st, xs_st, xs_st, xs_st, xs_st,
      dinv, W1, b1, W2)


def _tc_final(oparts, ys, dinv, b2):
    BM = 1024
    nb = NPAD // BM

    def body(op0_ref, op1_ref, ys_ref, dinv_ref, b2_ref, out_ref):
        acc = op0_ref[...] + op1_ref[...] + ys_ref[...]
        out_ref[...] = dinv_ref[...] * acc + b2_ref[...]

    return pl.pallas_call(
        body,
        grid=(nb,),
        in_specs=[
            pl.BlockSpec((BM, 1), lambda i: (i, 0)),
            pl.BlockSpec((BM, 1), lambda i: (nb + i, 0)),
            pl.BlockSpec((BM, 1), lambda i: (i, 0)),
            pl.BlockSpec((BM, 1), lambda i: (i, 0)),
            pl.BlockSpec((1, 1), lambda i: (0, 0)),
        ],
        out_specs=pl.BlockSpec((BM, 1), lambda i: (i, 0)),
        out_shape=jax.ShapeDtypeStruct((NPAD, 1), jnp.float32),
    )(oparts, oparts, ys, dinv, b2)


def kernel(x, edge_index, W1, b1, W2, b2):
    n = x.shape[0]
    e = edge_index.shape[1]
    src = edge_index[0].astype(jnp.int32)
    dst = edge_index[1].astype(jnp.int32)
    pad_id = NPAD - 1
    src = jnp.pad(src, (0, EPAD - e), constant_values=pad_id)
    dst = jnp.pad(dst, (0, EPAD - e), constant_values=pad_id)
    src2d = src.reshape(EPAD // CHUNK, CHUNK)
    dst2d = dst.reshape(EPAD // CHUNK, CHUNK)
    x_pad = jnp.pad(x, ((0, NPAD - n), (0, 0)))

    degp = _sc_degree(dst2d).reshape(NC * NPAD, 1)
    dinv, xs_st = _tc_prep(degp, x_pad)
    p0_st = _sc_rowscatter(src2d, dst2d, xs_st)
    ys = _tc_mlp(p0_st, xs_st, dinv, W1, b1.reshape(1, HID), W2)
    op = _sc_scalar_scatter(src2d, dst2d, ys.reshape(NPAD))
    out = _tc_final(op.reshape(NC * NPAD, 1), ys, dinv, b2.reshape(1, 1))
    return out[:n]
